# baseline (speedup 1.0000x reference)
import functools
import math

import numpy as np
import jax
import jax.numpy as jnp
from jax import lax
from jax.experimental import pallas as pl
from jax.experimental.pallas import tpu as pltpu
from jax.experimental.pallas import tpu_sc as plsc

L = 2048
DM = 768
H = 12
DK = 64
U = min(5 * int(np.ceil(np.log(L))), L)
EPS = 1e-6
NEG = float(np.float32(-3.0e38))


def _rotl32(x, d):
    return ((x << np.uint32(d)) | (x >> np.uint32(32 - d))).astype(np.uint32)


def _threefry2x32(k0, k1, x0, x1):
    rot = [(13, 15, 26, 6), (17, 29, 16, 24)]
    ks = [np.uint32(k0), np.uint32(k1),
          np.uint32(np.uint32(k0) ^ np.uint32(k1) ^ np.uint32(0x1BD11BDA))]
    x0 = (x0 + ks[0]).astype(np.uint32)
    x1 = (x1 + ks[1]).astype(np.uint32)
    for i in range(5):
        for r in rot[i % 2]:
            x0 = (x0 + x1).astype(np.uint32)
            x1 = _rotl32(x1, r)
            x1 = (x1 ^ x0).astype(np.uint32)
        x0 = (x0 + ks[(i + 1) % 3]).astype(np.uint32)
        x1 = (x1 + ks[(i + 2) % 3] + np.uint32(i + 1)).astype(np.uint32)
    return x0, x1


def _sample_indices() -> np.ndarray:
    b1, b2 = _threefry2x32(0, 42, np.zeros(2, np.uint32),
                           np.arange(2, dtype=np.uint32))
    n = L * U
    h1, h2 = _threefry2x32(b1[1], b2[1], np.zeros(n, np.uint32),
                           np.arange(n, dtype=np.uint32))
    bits = (h1 ^ h2).astype(np.uint32)
    return (bits % np.uint32(L)).astype(np.int32).reshape(L, U)


def _sample_counts_T() -> np.ndarray:
    idx = _sample_indices()
    cnt = np.zeros((L, L), np.float32)
    np.add.at(cnt, (np.arange(L)[:, None], idx), 1.0)
    return np.ascontiguousarray(cnt.T)


_CNT_T = _sample_counts_T()


def _proj_score_kernel(x_ref, wq_ref, wk_ref, wv_ref, cnt_ref,
                       q_out, k_out, v_out, m_out):
    x = x_ref[...].astype(jnp.bfloat16)
    wq = wq_ref[...].astype(jnp.bfloat16)
    wk = wk_ref[...].astype(jnp.bfloat16)
    wv = wv_ref[...].astype(jnp.bfloat16)
    q_all = jnp.dot(x, wq, preferred_element_type=jnp.float32)
    q_all = q_all * (1.0 / math.sqrt(DK))
    k_all = jnp.dot(x, wk, preferred_element_type=jnp.float32)
    v_all = jnp.dot(x, wv, preferred_element_type=jnp.float32)
    q16 = q_all.astype(jnp.bfloat16)
    k16 = k_all.astype(jnp.bfloat16)
    q_out[...] = q16
    k_out[...] = k16
    v_out[...] = v_all.astype(jnp.bfloat16)

    KB = 512
    m_rows = []
    for h in range(H):
        qh = q16[:, h * DK:(h + 1) * DK]
        kh = k16[:, h * DK:(h + 1) * DK]
        runmax = jnp.full((1, L), NEG, jnp.float32)
        runsum = jnp.zeros((1, L), jnp.float32)
        for b in range(L // KB):
            kb = kh[b * KB:(b + 1) * KB, :]
            s = jax.lax.dot_general(kb, qh, (((1,), (1,)), ((), ())),
                                    preferred_element_type=jnp.float32)
            cnt = cnt_ref[b * KB:(b + 1) * KB, :].astype(jnp.float32)
            runmax = jnp.maximum(
                runmax,
                jnp.max(jnp.where(cnt > 0, s, NEG), axis=0, keepdims=True))
            runsum = runsum + jnp.sum(s * cnt, axis=0, keepdims=True)
        m_rows.append(runmax - runsum * (1.0 / L))

    m_out[...] = jnp.concatenate(m_rows, axis=0)


_UPAD = 48


@functools.cache
def _make_sc_topk():
    mesh = plsc.VectorSubcoreMesh(core_axis_name="c", subcore_axis_name="s")
    ln = 16
    nch = L // ln

    @functools.partial(
        pl.kernel,
        mesh=mesh,
        out_type=jax.ShapeDtypeStruct((H, _UPAD), jnp.int32),
        compiler_params=pltpu.CompilerParams(needs_layout_passes=False),
        scratch_types=[
            pltpu.VMEM((L,), jnp.float32),
            pltpu.VMEM((_UPAD,), jnp.int32),
            pltpu.SemaphoreType.DMA,
        ],
    )
    def sc_topk(m_hbm, idx_hbm, mrow, idxbuf, sem):
        cid = lax.axis_index("c")
        sid = lax.axis_index("s")
        wid = sid * 2 + cid

        @pl.when(wid < H)
        def _():
            pltpu.sync_copy(m_hbm.at[wid], mrow)
            lane_iota = lax.iota(jnp.int32, ln)
            negv = jnp.full((ln,), NEG, jnp.float32)
            idxvec = jnp.zeros((ln,), jnp.int32)

            for r in range(U):
                def scanc(c, carry):
                    lm, ac = carry
                    v = mrow[pl.ds(c * ln, ln)]
                    better = v > lm
                    return (jnp.where(better, v, lm),
                            jnp.where(better, c, ac))

                lanemax, argc = lax.fori_loop(
                    0, nch, scanc,
                    (negv, jnp.zeros((ln,), jnp.int32)), unroll=8)
                gmax = lax.reduce_max(lanemax, (0,))
                cand = jnp.where(lanemax == gmax,
                                 argc * ln + lane_iota, 10**9)
                gidx = lax.reduce_min(cand, (0,))

                cbase = (gidx // ln) * ln
                v = mrow[pl.ds(cbase, ln)]
                v = jnp.where(lane_iota == gidx % ln, jnp.float32(NEG), v)
                mrow[pl.ds(cbase, ln)] = v

                idxvec = jnp.where(lane_iota == (r % ln), gidx, idxvec)
                if r % ln == ln - 1 or r == U - 1:
                    idxbuf[pl.ds((r // ln) * ln, ln)] = idxvec

            pltpu.sync_copy(idxbuf, idx_hbm.at[wid])

    return sc_topk


def _attn_apply_kernel(idx_sref, q_ref, k_ref, v_ref, ctx_ref):
    g = pl.program_id(0)
    iota = jax.lax.broadcasted_iota(jnp.int32, (1, L), 1)
    halves = []
    for j in range(2):
        q = q_ref[:, j * DK:(j + 1) * DK]
        k = k_ref[:, j * DK:(j + 1) * DK]
        v16 = v_ref[:, j * DK:(j + 1) * DK]
        v = v16.astype(jnp.float32)
        base = (2 * g + j) * U
        rows = [(iota == idx_sref[base + r]).astype(jnp.bfloat16)
                for r in range(U)]
        onehot = jnp.concatenate(rows, axis=0)

        q_sel = jnp.dot(onehot, q, preferred_element_type=jnp.float32)
        q_sel = q_sel.astype(jnp.bfloat16)
        scores = jax.lax.dot_general(q_sel, k, (((1,), (1,)), ((), ())),
                                     preferred_element_type=jnp.float32)
        smax = jnp.max(scores, axis=1, keepdims=True)
        e = jnp.exp(scores - smax)
        attn = (e / jnp.sum(e, axis=1, keepdims=True)).astype(jnp.bfloat16)
        upd = jnp.dot(attn, v16, preferred_element_type=jnp.float32)

        meanv = jnp.mean(v, axis=0, keepdims=True)
        delta = upd - meanv
        halves.append(
            jnp.broadcast_to(meanv, (L, DK)) + jax.lax.dot_general(
                onehot.astype(jnp.float32), delta, (((0,), (0,)), ((), ())),
                preferred_element_type=jnp.float32))
    ctx_ref[...] = jnp.concatenate(halves, axis=1)


def _out_kernel(ctx_ref, res_ref, wfc_ref, bfc_ref, g_ref, b_ref, o_ref):
    t = jnp.dot(ctx_ref[...].astype(jnp.bfloat16),
                wfc_ref[...].astype(jnp.bfloat16),
                preferred_element_type=jnp.float32)
    t = t + bfc_ref[...] + res_ref[...]
    mu = jnp.mean(t, axis=1, keepdims=True)
    d = t - mu
    var = jnp.mean(d * d, axis=1, keepdims=True)
    o_ref[...] = d * jax.lax.rsqrt(var + EPS) * g_ref[...] + b_ref[...]


def kernel(hidden_states, Wq, Wk, Wv, Wfc, bfc, gamma, beta):
    x = hidden_states.reshape(L, DM)
    cnt_t = jnp.asarray(_CNT_T).astype(jnp.bfloat16)

    q2, k2, v2, m2 = pl.pallas_call(
        _proj_score_kernel,
        out_shape=[
            jax.ShapeDtypeStruct((L, DM), jnp.bfloat16),
            jax.ShapeDtypeStruct((L, DM), jnp.bfloat16),
            jax.ShapeDtypeStruct((L, DM), jnp.bfloat16),
            jax.ShapeDtypeStruct((H, L), jnp.float32),
        ],
    )(x, Wq, Wk, Wv, cnt_t)

    idx = _make_sc_topk()(m2)[:, :U]

    ctx = pl.pallas_call(
        _attn_apply_kernel,
        grid_spec=pltpu.PrefetchScalarGridSpec(
            num_scalar_prefetch=1,
            grid=(H // 2,),
            in_specs=[
                pl.BlockSpec((L, 2 * DK), lambda g, idx_sref: (0, g)),
                pl.BlockSpec((L, 2 * DK), lambda g, idx_sref: (0, g)),
                pl.BlockSpec((L, 2 * DK), lambda g, idx_sref: (0, g)),
            ],
            out_specs=pl.BlockSpec((L, 2 * DK), lambda g, idx_sref: (0, g)),
        ),
        out_shape=jax.ShapeDtypeStruct((L, H * DK), jnp.float32),
    )(idx.reshape(H * U), q2, k2, v2)

    BL = 256
    out = pl.pallas_call(
        _out_kernel,
        grid=(L // BL,),
        in_specs=[
            pl.BlockSpec((BL, DM), lambda i: (i, 0)),
            pl.BlockSpec((BL, DM), lambda i: (i, 0)),
            pl.BlockSpec((DM, DM), lambda i: (0, 0)),
            pl.BlockSpec((1, DM), lambda i: (0, 0)),
            pl.BlockSpec((1, DM), lambda i: (0, 0)),
            pl.BlockSpec((1, DM), lambda i: (0, 0)),
        ],
        out_specs=pl.BlockSpec((BL, DM), lambda i: (i, 0)),
        out_shape=jax.ShapeDtypeStruct((L, DM), jnp.float32),
    )(ctx, x, Wfc, bfc.reshape(1, DM), gamma.reshape(1, DM),
      beta.reshape(1, DM))

    return out.reshape(1, L, DM)


# speedup vs baseline: 1.1645x; 1.1645x over previous
import math

import numpy as np
import jax
import jax.numpy as jnp
from jax.experimental import pallas as pl
from jax.experimental.pallas import tpu as pltpu

L = 2048
DM = 768
H = 12
DK = 64
U = min(5 * int(np.ceil(np.log(L))), L)
EPS = 1e-6
NEG = float(np.float32(-3.0e38))


def _rotl32(x, d):
    return ((x << np.uint32(d)) | (x >> np.uint32(32 - d))).astype(np.uint32)


def _threefry2x32(k0, k1, x0, x1):
    rot = [(13, 15, 26, 6), (17, 29, 16, 24)]
    ks = [np.uint32(k0), np.uint32(k1),
          np.uint32(np.uint32(k0) ^ np.uint32(k1) ^ np.uint32(0x1BD11BDA))]
    x0 = (x0 + ks[0]).astype(np.uint32)
    x1 = (x1 + ks[1]).astype(np.uint32)
    for i in range(5):
        for r in rot[i % 2]:
            x0 = (x0 + x1).astype(np.uint32)
            x1 = _rotl32(x1, r)
            x1 = (x1 ^ x0).astype(np.uint32)
        x0 = (x0 + ks[(i + 1) % 3]).astype(np.uint32)
        x1 = (x1 + ks[(i + 2) % 3] + np.uint32(i + 1)).astype(np.uint32)
    return x0, x1


def _sample_indices() -> np.ndarray:
    b1, b2 = _threefry2x32(0, 42, np.zeros(2, np.uint32),
                           np.arange(2, dtype=np.uint32))
    n = L * U
    h1, h2 = _threefry2x32(b1[1], b2[1], np.zeros(n, np.uint32),
                           np.arange(n, dtype=np.uint32))
    bits = (h1 ^ h2).astype(np.uint32)
    return (bits % np.uint32(L)).astype(np.int32).reshape(L, U)


def _sample_counts_T() -> np.ndarray:
    idx = _sample_indices()
    cnt = np.zeros((L, L), np.float32)
    np.add.at(cnt, (np.arange(L)[:, None], idx), 1.0)
    return np.ascontiguousarray(cnt.T)


_CNT_T = _sample_counts_T()


def _proj_score_topk_kernel(x_ref, wq_ref, wk_ref, wv_ref, cnt_ref,
                            q_out, k_out, v_out, idx_ref):
    x = x_ref[...].astype(jnp.bfloat16)
    wq = wq_ref[...].astype(jnp.bfloat16)
    wk = wk_ref[...].astype(jnp.bfloat16)
    wv = wv_ref[...].astype(jnp.bfloat16)
    q_all = jnp.dot(x, wq, preferred_element_type=jnp.float32)
    q_all = q_all * (1.0 / math.sqrt(DK))
    k_all = jnp.dot(x, wk, preferred_element_type=jnp.float32)
    v_all = jnp.dot(x, wv, preferred_element_type=jnp.float32)
    q16 = q_all.astype(jnp.bfloat16)
    k16 = k_all.astype(jnp.bfloat16)
    q_out[...] = q16
    k_out[...] = k16
    v_out[...] = v_all.astype(jnp.bfloat16)

    KB = 512
    m_rows = []
    for h in range(H):
        qh = q16[:, h * DK:(h + 1) * DK]
        kh = k16[:, h * DK:(h + 1) * DK]
        runmax = jnp.full((1, L), NEG, jnp.float32)
        runsum = jnp.zeros((1, L), jnp.float32)
        for b in range(L // KB):
            kb = kh[b * KB:(b + 1) * KB, :]
            s = jax.lax.dot_general(kb, qh, (((1,), (1,)), ((), ())),
                                    preferred_element_type=jnp.float32)
            cnt = cnt_ref[b * KB:(b + 1) * KB, :].astype(jnp.float32)
            runmax = jnp.maximum(
                runmax,
                jnp.max(jnp.where(cnt > 0, s, NEG), axis=0, keepdims=True))
            runsum = runsum + jnp.sum(s * cnt, axis=0, keepdims=True)
        m_rows.append(runmax - runsum * (1.0 / L))

    mv = jnp.concatenate(m_rows, axis=0)
    iota = jax.lax.broadcasted_iota(jnp.int32, (H, L), 1)
    for r in range(U):
        mx = jnp.max(mv, axis=1, keepdims=True)
        amin = jnp.min(jnp.where(mv == mx, iota, L), axis=1, keepdims=True)
        idx_ref[:, r:r + 1] = amin
        mv = jnp.where(iota == amin, NEG, mv)


def _attn_apply_kernel(idx_sref, q_ref, k_ref, v_ref, ctx_ref):
    g = pl.program_id(0)
    iota = jax.lax.broadcasted_iota(jnp.int32, (1, L), 1)
    halves = []
    for j in range(2):
        q = q_ref[:, j * DK:(j + 1) * DK]
        k = k_ref[:, j * DK:(j + 1) * DK]
        v16 = v_ref[:, j * DK:(j + 1) * DK]
        v = v16.astype(jnp.float32)
        base = (2 * g + j) * U
        rows = [(iota == idx_sref[base + r]).astype(jnp.bfloat16)
                for r in range(U)]
        onehot = jnp.concatenate(rows, axis=0)

        q_sel = jnp.dot(onehot, q, preferred_element_type=jnp.float32)
        q_sel = q_sel.astype(jnp.bfloat16)
        scores = jax.lax.dot_general(q_sel, k, (((1,), (1,)), ((), ())),
                                     preferred_element_type=jnp.float32)
        smax = jnp.max(scores, axis=1, keepdims=True)
        e = jnp.exp(scores - smax)
        attn = (e / jnp.sum(e, axis=1, keepdims=True)).astype(jnp.bfloat16)
        upd = jnp.dot(attn, v16, preferred_element_type=jnp.float32)

        meanv = jnp.mean(v, axis=0, keepdims=True)
        delta = upd - meanv
        halves.append(
            jnp.broadcast_to(meanv, (L, DK)) + jax.lax.dot_general(
                onehot.astype(jnp.float32), delta, (((0,), (0,)), ((), ())),
                preferred_element_type=jnp.float32))
    ctx_ref[...] = jnp.concatenate(halves, axis=1)


def _out_kernel(ctx_ref, res_ref, wfc_ref, bfc_ref, g_ref, b_ref, o_ref):
    t = jnp.dot(ctx_ref[...].astype(jnp.bfloat16),
                wfc_ref[...].astype(jnp.bfloat16),
                preferred_element_type=jnp.float32)
    t = t + bfc_ref[...] + res_ref[...]
    mu = jnp.mean(t, axis=1, keepdims=True)
    d = t - mu
    var = jnp.mean(d * d, axis=1, keepdims=True)
    o_ref[...] = d * jax.lax.rsqrt(var + EPS) * g_ref[...] + b_ref[...]


def kernel(hidden_states, Wq, Wk, Wv, Wfc, bfc, gamma, beta):
    x = hidden_states.reshape(L, DM)
    cnt_t = jnp.asarray(_CNT_T).astype(jnp.bfloat16)

    q2, k2, v2, idx = pl.pallas_call(
        _proj_score_topk_kernel,
        out_shape=[
            jax.ShapeDtypeStruct((L, DM), jnp.bfloat16),
            jax.ShapeDtypeStruct((L, DM), jnp.bfloat16),
            jax.ShapeDtypeStruct((L, DM), jnp.bfloat16),
            jax.ShapeDtypeStruct((H, U), jnp.int32),
        ],
    )(x, Wq, Wk, Wv, cnt_t)

    ctx = pl.pallas_call(
        _attn_apply_kernel,
        grid_spec=pltpu.PrefetchScalarGridSpec(
            num_scalar_prefetch=1,
            grid=(H // 2,),
            in_specs=[
                pl.BlockSpec((L, 2 * DK), lambda g, idx_sref: (0, g)),
                pl.BlockSpec((L, 2 * DK), lambda g, idx_sref: (0, g)),
                pl.BlockSpec((L, 2 * DK), lambda g, idx_sref: (0, g)),
            ],
            out_specs=pl.BlockSpec((L, 2 * DK), lambda g, idx_sref: (0, g)),
        ),
        out_shape=jax.ShapeDtypeStruct((L, H * DK), jnp.float32),
    )(idx.reshape(H * U), q2, k2, v2)

    BL = 256
    out = pl.pallas_call(
        _out_kernel,
        grid=(L // BL,),
        in_specs=[
            pl.BlockSpec((BL, DM), lambda i: (i, 0)),
            pl.BlockSpec((BL, DM), lambda i: (i, 0)),
            pl.BlockSpec((DM, DM), lambda i: (0, 0)),
            pl.BlockSpec((1, DM), lambda i: (0, 0)),
            pl.BlockSpec((1, DM), lambda i: (0, 0)),
            pl.BlockSpec((1, DM), lambda i: (0, 0)),
        ],
        out_specs=pl.BlockSpec((BL, DM), lambda i: (i, 0)),
        out_shape=jax.ShapeDtypeStruct((L, DM), jnp.float32),
    )(ctx, x, Wfc, bfc.reshape(1, DM), gamma.reshape(1, DM),
      beta.reshape(1, DM))

    return out.reshape(1, L, DM)


# speedup vs baseline: 1.1774x; 1.0111x over previous
import math

import numpy as np
import jax
import jax.numpy as jnp
from jax.experimental import pallas as pl
from jax.experimental.pallas import tpu as pltpu

L = 2048
DM = 768
H = 12
DK = 64
U = min(5 * int(np.ceil(np.log(L))), L)
EPS = 1e-6
NEG = float(np.float32(-3.0e38))


def _rotl32(x, d):
    return ((x << np.uint32(d)) | (x >> np.uint32(32 - d))).astype(np.uint32)


def _threefry2x32(k0, k1, x0, x1):
    rot = [(13, 15, 26, 6), (17, 29, 16, 24)]
    ks = [np.uint32(k0), np.uint32(k1),
          np.uint32(np.uint32(k0) ^ np.uint32(k1) ^ np.uint32(0x1BD11BDA))]
    x0 = (x0 + ks[0]).astype(np.uint32)
    x1 = (x1 + ks[1]).astype(np.uint32)
    for i in range(5):
        for r in rot[i % 2]:
            x0 = (x0 + x1).astype(np.uint32)
            x1 = _rotl32(x1, r)
            x1 = (x1 ^ x0).astype(np.uint32)
        x0 = (x0 + ks[(i + 1) % 3]).astype(np.uint32)
        x1 = (x1 + ks[(i + 2) % 3] + np.uint32(i + 1)).astype(np.uint32)
    return x0, x1


def _sample_indices() -> np.ndarray:
    b1, b2 = _threefry2x32(0, 42, np.zeros(2, np.uint32),
                           np.arange(2, dtype=np.uint32))
    n = L * U
    h1, h2 = _threefry2x32(b1[1], b2[1], np.zeros(n, np.uint32),
                           np.arange(n, dtype=np.uint32))
    bits = (h1 ^ h2).astype(np.uint32)
    return (bits % np.uint32(L)).astype(np.int32).reshape(L, U)


def _sample_counts_T() -> np.ndarray:
    idx = _sample_indices()
    cnt = np.zeros((L, L), np.float32)
    np.add.at(cnt, (np.arange(L)[:, None], idx), 1.0)
    return np.ascontiguousarray(cnt.T)


_CNT_T = _sample_counts_T()


def _proj_score_topk_kernel(x_ref, wq_ref, wk_ref, wv_ref, cnt_ref,
                            q_out, k_out, v_out, idx_ref):
    x = x_ref[...].astype(jnp.bfloat16)
    wq = wq_ref[...].astype(jnp.bfloat16)
    wk = wk_ref[...].astype(jnp.bfloat16)
    wv = wv_ref[...].astype(jnp.bfloat16)
    q_all = jnp.dot(x, wq, preferred_element_type=jnp.float32)
    q_all = q_all * (1.0 / math.sqrt(DK))
    k_all = jnp.dot(x, wk, preferred_element_type=jnp.float32)
    v_all = jnp.dot(x, wv, preferred_element_type=jnp.float32)
    q16 = q_all.astype(jnp.bfloat16)
    k16 = k_all.astype(jnp.bfloat16)
    q_out[...] = q16
    k_out[...] = k16
    v_out[...] = v_all.astype(jnp.bfloat16)

    KB = 1024
    m_rows = []
    for h in range(H):
        qh = q16[:, h * DK:(h + 1) * DK]
        kh = k16[:, h * DK:(h + 1) * DK]
        runmax = jnp.full((1, L), NEG, jnp.float32)
        runsum = jnp.zeros((1, L), jnp.float32)
        for b in range(L // KB):
            kb = kh[b * KB:(b + 1) * KB, :]
            s = jax.lax.dot_general(kb, qh, (((1,), (1,)), ((), ())),
                                    preferred_element_type=jnp.float32)
            cnt = cnt_ref[b * KB:(b + 1) * KB, :].astype(jnp.float32)
            runmax = jnp.maximum(
                runmax,
                jnp.max(jnp.where(cnt > 0, s, NEG), axis=0, keepdims=True))
            runsum = runsum + jnp.sum(s * cnt, axis=0, keepdims=True)
        m_rows.append(runmax - runsum * (1.0 / L))

    mv = jnp.concatenate(m_rows, axis=0)
    iota = jax.lax.broadcasted_iota(jnp.int32, (H, L), 1)
    for r in range(U):
        mx = jnp.max(mv, axis=1, keepdims=True)
        amin = jnp.min(jnp.where(mv == mx, iota, L), axis=1, keepdims=True)
        idx_ref[:, r:r + 1] = amin
        mv = jnp.where(iota == amin, NEG, mv)


def _attn_apply_kernel(idx_sref, q_ref, k_ref, v_ref, ctx_ref):
    g = pl.program_id(0)
    iota = jax.lax.broadcasted_iota(jnp.int32, (1, L), 1)
    halves = []
    for j in range(2):
        q = q_ref[:, j * DK:(j + 1) * DK]
        k = k_ref[:, j * DK:(j + 1) * DK]
        v16 = v_ref[:, j * DK:(j + 1) * DK]
        v = v16.astype(jnp.float32)
        base = (2 * g + j) * U
        rows = [(iota == idx_sref[base + r]).astype(jnp.bfloat16)
                for r in range(U)]
        onehot = jnp.concatenate(rows, axis=0)

        q_sel = jnp.dot(onehot, q, preferred_element_type=jnp.float32)
        q_sel = q_sel.astype(jnp.bfloat16)
        scores = jax.lax.dot_general(q_sel, k, (((1,), (1,)), ((), ())),
                                     preferred_element_type=jnp.float32)
        smax = jnp.max(scores, axis=1, keepdims=True)
        e = jnp.exp(scores - smax)
        attn = (e / jnp.sum(e, axis=1, keepdims=True)).astype(jnp.bfloat16)
        upd = jnp.dot(attn, v16, preferred_element_type=jnp.float32)

        meanv = jnp.mean(v, axis=0, keepdims=True)
        delta = upd - meanv
        halves.append(
            jnp.broadcast_to(meanv, (L, DK)) + jax.lax.dot_general(
                onehot.astype(jnp.float32), delta, (((0,), (0,)), ((), ())),
                preferred_element_type=jnp.float32))
    ctx_ref[...] = jnp.concatenate(halves, axis=1)


def _out_kernel(ctx_ref, res_ref, wfc_ref, bfc_ref, g_ref, b_ref, o_ref):
    t = jnp.dot(ctx_ref[...].astype(jnp.bfloat16),
                wfc_ref[...].astype(jnp.bfloat16),
                preferred_element_type=jnp.float32)
    t = t + bfc_ref[...] + res_ref[...]
    mu = jnp.mean(t, axis=1, keepdims=True)
    d = t - mu
    var = jnp.mean(d * d, axis=1, keepdims=True)
    o_ref[...] = d * jax.lax.rsqrt(var + EPS) * g_ref[...] + b_ref[...]


def kernel(hidden_states, Wq, Wk, Wv, Wfc, bfc, gamma, beta):
    x = hidden_states.reshape(L, DM)
    cnt_t = jnp.asarray(_CNT_T).astype(jnp.bfloat16)

    q2, k2, v2, idx = pl.pallas_call(
        _proj_score_topk_kernel,
        out_shape=[
            jax.ShapeDtypeStruct((L, DM), jnp.bfloat16),
            jax.ShapeDtypeStruct((L, DM), jnp.bfloat16),
            jax.ShapeDtypeStruct((L, DM), jnp.bfloat16),
            jax.ShapeDtypeStruct((H, U), jnp.int32),
        ],
    )(x, Wq, Wk, Wv, cnt_t)

    ctx = pl.pallas_call(
        _attn_apply_kernel,
        grid_spec=pltpu.PrefetchScalarGridSpec(
            num_scalar_prefetch=1,
            grid=(H // 2,),
            in_specs=[
                pl.BlockSpec((L, 2 * DK), lambda g, idx_sref: (0, g)),
                pl.BlockSpec((L, 2 * DK), lambda g, idx_sref: (0, g)),
                pl.BlockSpec((L, 2 * DK), lambda g, idx_sref: (0, g)),
            ],
            out_specs=pl.BlockSpec((L, 2 * DK), lambda g, idx_sref: (0, g)),
        ),
        out_shape=jax.ShapeDtypeStruct((L, H * DK), jnp.float32),
    )(idx.reshape(H * U), q2, k2, v2)

    BL = 256
    out = pl.pallas_call(
        _out_kernel,
        grid=(L // BL,),
        in_specs=[
            pl.BlockSpec((BL, DM), lambda i: (i, 0)),
            pl.BlockSpec((BL, DM), lambda i: (i, 0)),
            pl.BlockSpec((DM, DM), lambda i: (0, 0)),
            pl.BlockSpec((1, DM), lambda i: (0, 0)),
            pl.BlockSpec((1, DM), lambda i: (0, 0)),
            pl.BlockSpec((1, DM), lambda i: (0, 0)),
        ],
        out_specs=pl.BlockSpec((BL, DM), lambda i: (i, 0)),
        out_shape=jax.ShapeDtypeStruct((L, DM), jnp.float32),
    )(ctx, x, Wfc, bfc.reshape(1, DM), gamma.reshape(1, DM),
      beta.reshape(1, DM))

    return out.reshape(1, L, DM)


# speedup vs baseline: 1.1814x; 1.0033x over previous
import math

import numpy as np
import jax
import jax.numpy as jnp
from jax.experimental import pallas as pl
from jax.experimental.pallas import tpu as pltpu

L = 2048
DM = 768
H = 12
DK = 64
U = min(5 * int(np.ceil(np.log(L))), L)
EPS = 1e-6
NEG = float(np.float32(-3.0e38))


def _rotl32(x, d):
    return ((x << np.uint32(d)) | (x >> np.uint32(32 - d))).astype(np.uint32)


def _threefry2x32(k0, k1, x0, x1):
    rot = [(13, 15, 26, 6), (17, 29, 16, 24)]
    ks = [np.uint32(k0), np.uint32(k1),
          np.uint32(np.uint32(k0) ^ np.uint32(k1) ^ np.uint32(0x1BD11BDA))]
    x0 = (x0 + ks[0]).astype(np.uint32)
    x1 = (x1 + ks[1]).astype(np.uint32)
    for i in range(5):
        for r in rot[i % 2]:
            x0 = (x0 + x1).astype(np.uint32)
            x1 = _rotl32(x1, r)
            x1 = (x1 ^ x0).astype(np.uint32)
        x0 = (x0 + ks[(i + 1) % 3]).astype(np.uint32)
        x1 = (x1 + ks[(i + 2) % 3] + np.uint32(i + 1)).astype(np.uint32)
    return x0, x1


def _sample_indices() -> np.ndarray:
    b1, b2 = _threefry2x32(0, 42, np.zeros(2, np.uint32),
                           np.arange(2, dtype=np.uint32))
    n = L * U
    h1, h2 = _threefry2x32(b1[1], b2[1], np.zeros(n, np.uint32),
                           np.arange(n, dtype=np.uint32))
    bits = (h1 ^ h2).astype(np.uint32)
    return (bits % np.uint32(L)).astype(np.int32).reshape(L, U)


def _sample_counts_T() -> np.ndarray:
    idx = _sample_indices()
    cnt = np.zeros((L, L), np.float32)
    np.add.at(cnt, (np.arange(L)[:, None], idx), 1.0)
    return np.ascontiguousarray(cnt.T)


_CNT_T = _sample_counts_T()


def _proj_score_topk_kernel(x_ref, wq_ref, wk_ref, wv_ref, cnt_ref,
                            q_out, k_out, v_out, idx_ref):
    x = x_ref[...].astype(jnp.bfloat16)
    wq = wq_ref[...].astype(jnp.bfloat16)
    wk = wk_ref[...].astype(jnp.bfloat16)
    wv = wv_ref[...].astype(jnp.bfloat16)
    q_all = jnp.dot(x, wq, preferred_element_type=jnp.float32)
    q_all = q_all * (1.0 / math.sqrt(DK))
    k_all = jnp.dot(x, wk, preferred_element_type=jnp.float32)
    v_all = jnp.dot(x, wv, preferred_element_type=jnp.float32)
    q16 = q_all.astype(jnp.bfloat16)
    k16 = k_all.astype(jnp.bfloat16)
    q_out[...] = q16
    k_out[...] = k16
    v_out[...] = v_all.astype(jnp.bfloat16)

    KB = 2048
    m_rows = []
    for h in range(H):
        qh = q16[:, h * DK:(h + 1) * DK]
        kh = k16[:, h * DK:(h + 1) * DK]
        runmax = jnp.full((1, L), NEG, jnp.float32)
        runsum = jnp.zeros((1, L), jnp.float32)
        for b in range(L // KB):
            kb = kh[b * KB:(b + 1) * KB, :]
            s = jax.lax.dot_general(kb, qh, (((1,), (1,)), ((), ())),
                                    preferred_element_type=jnp.float32)
            cnt = cnt_ref[b * KB:(b + 1) * KB, :].astype(jnp.float32)
            runmax = jnp.maximum(
                runmax,
                jnp.max(jnp.where(cnt > 0, s, NEG), axis=0, keepdims=True))
            runsum = runsum + jnp.sum(s * cnt, axis=0, keepdims=True)
        m_rows.append(runmax - runsum * (1.0 / L))

    mv = jnp.concatenate(m_rows, axis=0)
    iota = jax.lax.broadcasted_iota(jnp.int32, (H, L), 1)
    for r in range(U):
        mx = jnp.max(mv, axis=1, keepdims=True)
        amin = jnp.min(jnp.where(mv == mx, iota, L), axis=1, keepdims=True)
        idx_ref[:, r:r + 1] = amin
        mv = jnp.where(iota == amin, NEG, mv)


def _attn_apply_kernel(idx_sref, q_ref, k_ref, v_ref, ctx_ref):
    g = pl.program_id(0)
    iota = jax.lax.broadcasted_iota(jnp.int32, (1, L), 1)
    halves = []
    for j in range(2):
        q = q_ref[:, j * DK:(j + 1) * DK]
        k = k_ref[:, j * DK:(j + 1) * DK]
        v16 = v_ref[:, j * DK:(j + 1) * DK]
        v = v16.astype(jnp.float32)
        base = (2 * g + j) * U
        rows = [(iota == idx_sref[base + r]).astype(jnp.bfloat16)
                for r in range(U)]
        onehot = jnp.concatenate(rows, axis=0)

        q_sel = jnp.dot(onehot, q, preferred_element_type=jnp.float32)
        q_sel = q_sel.astype(jnp.bfloat16)
        scores = jax.lax.dot_general(q_sel, k, (((1,), (1,)), ((), ())),
                                     preferred_element_type=jnp.float32)
        smax = jnp.max(scores, axis=1, keepdims=True)
        e = jnp.exp(scores - smax)
        attn = (e / jnp.sum(e, axis=1, keepdims=True)).astype(jnp.bfloat16)
        upd = jnp.dot(attn, v16, preferred_element_type=jnp.float32)

        meanv = jnp.mean(v, axis=0, keepdims=True)
        delta = upd - meanv
        halves.append(
            jnp.broadcast_to(meanv, (L, DK)) + jax.lax.dot_general(
                onehot.astype(jnp.float32), delta, (((0,), (0,)), ((), ())),
                preferred_element_type=jnp.float32))
    ctx_ref[...] = jnp.concatenate(halves, axis=1)


def _out_kernel(ctx_ref, res_ref, wfc_ref, bfc_ref, g_ref, b_ref, o_ref):
    t = jnp.dot(ctx_ref[...].astype(jnp.bfloat16),
                wfc_ref[...].astype(jnp.bfloat16),
                preferred_element_type=jnp.float32)
    t = t + bfc_ref[...] + res_ref[...]
    mu = jnp.mean(t, axis=1, keepdims=True)
    d = t - mu
    var = jnp.mean(d * d, axis=1, keepdims=True)
    o_ref[...] = d * jax.lax.rsqrt(var + EPS) * g_ref[...] + b_ref[...]


def kernel(hidden_states, Wq, Wk, Wv, Wfc, bfc, gamma, beta):
    x = hidden_states.reshape(L, DM)
    cnt_t = jnp.asarray(_CNT_T).astype(jnp.bfloat16)

    q2, k2, v2, idx = pl.pallas_call(
        _proj_score_topk_kernel,
        out_shape=[
            jax.ShapeDtypeStruct((L, DM), jnp.bfloat16),
            jax.ShapeDtypeStruct((L, DM), jnp.bfloat16),
            jax.ShapeDtypeStruct((L, DM), jnp.bfloat16),
            jax.ShapeDtypeStruct((H, U), jnp.int32),
        ],
    )(x, Wq, Wk, Wv, cnt_t)

    ctx = pl.pallas_call(
        _attn_apply_kernel,
        grid_spec=pltpu.PrefetchScalarGridSpec(
            num_scalar_prefetch=1,
            grid=(H // 2,),
            in_specs=[
                pl.BlockSpec((L, 2 * DK), lambda g, idx_sref: (0, g)),
                pl.BlockSpec((L, 2 * DK), lambda g, idx_sref: (0, g)),
                pl.BlockSpec((L, 2 * DK), lambda g, idx_sref: (0, g)),
            ],
            out_specs=pl.BlockSpec((L, 2 * DK), lambda g, idx_sref: (0, g)),
        ),
        out_shape=jax.ShapeDtypeStruct((L, H * DK), jnp.float32),
    )(idx.reshape(H * U), q2, k2, v2)

    BL = 256
    out = pl.pallas_call(
        _out_kernel,
        grid=(L // BL,),
        in_specs=[
            pl.BlockSpec((BL, DM), lambda i: (i, 0)),
            pl.BlockSpec((BL, DM), lambda i: (i, 0)),
            pl.BlockSpec((DM, DM), lambda i: (0, 0)),
            pl.BlockSpec((1, DM), lambda i: (0, 0)),
            pl.BlockSpec((1, DM), lambda i: (0, 0)),
            pl.BlockSpec((1, DM), lambda i: (0, 0)),
        ],
        out_specs=pl.BlockSpec((BL, DM), lambda i: (i, 0)),
        out_shape=jax.ShapeDtypeStruct((L, DM), jnp.float32),
    )(ctx, x, Wfc, bfc.reshape(1, DM), gamma.reshape(1, DM),
      beta.reshape(1, DM))

    return out.reshape(1, L, DM)


# speedup vs baseline: 1.2023x; 1.0177x over previous
import math

import numpy as np
import jax
import jax.numpy as jnp
from jax.experimental import pallas as pl
from jax.experimental.pallas import tpu as pltpu

L = 2048
DM = 768
H = 12
DK = 64
U = min(5 * int(np.ceil(np.log(L))), L)
EPS = 1e-6
NEG = float(np.float32(-3.0e38))


def _rotl32(x, d):
    return ((x << np.uint32(d)) | (x >> np.uint32(32 - d))).astype(np.uint32)


def _threefry2x32(k0, k1, x0, x1):
    rot = [(13, 15, 26, 6), (17, 29, 16, 24)]
    ks = [np.uint32(k0), np.uint32(k1),
          np.uint32(np.uint32(k0) ^ np.uint32(k1) ^ np.uint32(0x1BD11BDA))]
    x0 = (x0 + ks[0]).astype(np.uint32)
    x1 = (x1 + ks[1]).astype(np.uint32)
    for i in range(5):
        for r in rot[i % 2]:
            x0 = (x0 + x1).astype(np.uint32)
            x1 = _rotl32(x1, r)
            x1 = (x1 ^ x0).astype(np.uint32)
        x0 = (x0 + ks[(i + 1) % 3]).astype(np.uint32)
        x1 = (x1 + ks[(i + 2) % 3] + np.uint32(i + 1)).astype(np.uint32)
    return x0, x1


def _sample_indices() -> np.ndarray:
    b1, b2 = _threefry2x32(0, 42, np.zeros(2, np.uint32),
                           np.arange(2, dtype=np.uint32))
    n = L * U
    h1, h2 = _threefry2x32(b1[1], b2[1], np.zeros(n, np.uint32),
                           np.arange(n, dtype=np.uint32))
    bits = (h1 ^ h2).astype(np.uint32)
    return (bits % np.uint32(L)).astype(np.int32).reshape(L, U)


def _sample_counts_T() -> np.ndarray:
    idx = _sample_indices()
    cnt = np.zeros((L, L), np.float32)
    np.add.at(cnt, (np.arange(L)[:, None], idx), 1.0)
    return np.ascontiguousarray(cnt.T)


_CNT_T = _sample_counts_T()


def _proj_score_topk_kernel(x_ref, wq_ref, wk_ref, wv_ref, cnt_ref,
                            q_out, k_out, v_out, idx_ref):
    x = x_ref[...].astype(jnp.bfloat16)
    wq = wq_ref[...].astype(jnp.bfloat16)
    wk = wk_ref[...].astype(jnp.bfloat16)
    wv = wv_ref[...].astype(jnp.bfloat16)
    q_all = jnp.dot(x, wq, preferred_element_type=jnp.float32)
    q_all = q_all * (1.0 / math.sqrt(DK))
    k_all = jnp.dot(x, wk, preferred_element_type=jnp.float32)
    v_all = jnp.dot(x, wv, preferred_element_type=jnp.float32)
    q16 = q_all.astype(jnp.bfloat16)
    k16 = k_all.astype(jnp.bfloat16)
    q_out[...] = q16
    k_out[...] = k16
    v_out[...] = v_all.astype(jnp.bfloat16)

    KB = 2048
    m_rows = []
    for h in range(H):
        qh = q16[:, h * DK:(h + 1) * DK]
        kh = k16[:, h * DK:(h + 1) * DK]
        runmax = jnp.full((1, L), NEG, jnp.float32)
        runsum = jnp.zeros((1, L), jnp.float32)
        for b in range(L // KB):
            kb = kh[b * KB:(b + 1) * KB, :]
            s = jax.lax.dot_general(kb, qh, (((1,), (1,)), ((), ())),
                                    preferred_element_type=jnp.float32)
            cnt = cnt_ref[b * KB:(b + 1) * KB, :].astype(jnp.float32)
            runmax = jnp.maximum(
                runmax,
                jnp.max(jnp.where(cnt > 0, s, NEG), axis=0, keepdims=True))
            runsum = runsum + jnp.sum(s * cnt, axis=0, keepdims=True)
        m_rows.append(runmax - runsum * (1.0 / L))

    mv = jnp.concatenate(m_rows, axis=0)
    iota = jax.lax.broadcasted_iota(jnp.int32, (H, L), 1)
    for r in range(U):
        mx = jnp.max(mv, axis=1, keepdims=True)
        amin = jnp.min(jnp.where(mv == mx, iota, L), axis=1, keepdims=True)
        idx_ref[:, r:r + 1] = amin
        mv = jnp.where(iota == amin, NEG, mv)


def _attn_apply_kernel(idx_sref, q_ref, k_ref, v_ref, ctx_ref):
    g = pl.program_id(0)
    iota = jax.lax.broadcasted_iota(jnp.int32, (1, L), 1)
    halves = []
    for j in range(2):
        q = q_ref[:, j * DK:(j + 1) * DK]
        k = k_ref[:, j * DK:(j + 1) * DK]
        v16 = v_ref[:, j * DK:(j + 1) * DK]
        v = v16.astype(jnp.float32)
        base = (2 * g + j) * U
        rows = [(iota == idx_sref[base + r]).astype(jnp.bfloat16)
                for r in range(U)]
        onehot = jnp.concatenate(rows, axis=0)

        q_sel = jnp.dot(onehot, q, preferred_element_type=jnp.float32)
        q_sel = q_sel.astype(jnp.bfloat16)
        scores = jax.lax.dot_general(q_sel, k, (((1,), (1,)), ((), ())),
                                     preferred_element_type=jnp.float32)
        smax = jnp.max(scores, axis=1, keepdims=True)
        e = jnp.exp(scores - smax)
        attn = (e / jnp.sum(e, axis=1, keepdims=True)).astype(jnp.bfloat16)
        upd = jnp.dot(attn, v16, preferred_element_type=jnp.float32)

        meanv = jnp.mean(v, axis=0, keepdims=True)
        delta = upd - meanv
        halves.append(
            jnp.broadcast_to(meanv, (L, DK)) + jax.lax.dot_general(
                onehot.astype(jnp.float32), delta, (((0,), (0,)), ((), ())),
                preferred_element_type=jnp.float32))
    ctx_ref[...] = jnp.concatenate(halves, axis=1)


def _out_kernel(ctx_ref, res_ref, wfc_ref, bfc_ref, g_ref, b_ref, o_ref):
    t = jnp.dot(ctx_ref[...].astype(jnp.bfloat16),
                wfc_ref[...].astype(jnp.bfloat16),
                preferred_element_type=jnp.float32)
    t = t + bfc_ref[...] + res_ref[...]
    mu = jnp.mean(t, axis=1, keepdims=True)
    d = t - mu
    var = jnp.mean(d * d, axis=1, keepdims=True)
    o_ref[...] = d * jax.lax.rsqrt(var + EPS) * g_ref[...] + b_ref[...]


def kernel(hidden_states, Wq, Wk, Wv, Wfc, bfc, gamma, beta):
    x = hidden_states.reshape(L, DM)
    cnt_t = jnp.asarray(_CNT_T).astype(jnp.bfloat16)

    q2, k2, v2, idx = pl.pallas_call(
        _proj_score_topk_kernel,
        out_shape=[
            jax.ShapeDtypeStruct((L, DM), jnp.bfloat16),
            jax.ShapeDtypeStruct((L, DM), jnp.bfloat16),
            jax.ShapeDtypeStruct((L, DM), jnp.bfloat16),
            jax.ShapeDtypeStruct((H, U), jnp.int32),
        ],
    )(x, Wq, Wk, Wv, cnt_t)

    ctx = pl.pallas_call(
        _attn_apply_kernel,
        grid_spec=pltpu.PrefetchScalarGridSpec(
            num_scalar_prefetch=1,
            grid=(H // 2,),
            in_specs=[
                pl.BlockSpec((L, 2 * DK), lambda g, idx_sref: (0, g)),
                pl.BlockSpec((L, 2 * DK), lambda g, idx_sref: (0, g)),
                pl.BlockSpec((L, 2 * DK), lambda g, idx_sref: (0, g)),
            ],
            out_specs=pl.BlockSpec((L, 2 * DK), lambda g, idx_sref: (0, g)),
        ),
        out_shape=jax.ShapeDtypeStruct((L, H * DK), jnp.float32),
    )(idx.reshape(H * U), q2, k2, v2)

    BL = 512
    out = pl.pallas_call(
        _out_kernel,
        grid=(L // BL,),
        in_specs=[
            pl.BlockSpec((BL, DM), lambda i: (i, 0)),
            pl.BlockSpec((BL, DM), lambda i: (i, 0)),
            pl.BlockSpec((DM, DM), lambda i: (0, 0)),
            pl.BlockSpec((1, DM), lambda i: (0, 0)),
            pl.BlockSpec((1, DM), lambda i: (0, 0)),
            pl.BlockSpec((1, DM), lambda i: (0, 0)),
        ],
        out_specs=pl.BlockSpec((BL, DM), lambda i: (i, 0)),
        out_shape=jax.ShapeDtypeStruct((L, DM), jnp.float32),
    )(ctx, x, Wfc, bfc.reshape(1, DM), gamma.reshape(1, DM),
      beta.reshape(1, DM))

    return out.reshape(1, L, DM)
